# baseline (speedup 1.0000x reference)
import functools

import jax
import jax.numpy as jnp
from jax import lax
from jax.experimental import pallas as pl
from jax.experimental.pallas import tpu as pltpu
from jax.experimental.pallas import tpu_sc as plsc

_NC = 2
_NS = 16
_KH = 5


@functools.partial(jax.jit, static_argnames=("H", "B"))
def _gather(idx_tiles, table_t, H, B):
    D, V = table_t.shape
    d_per_core = D // _NC
    b_chunk = B // _NS
    nq = H // _KH
    cs = _KH * b_chunk
    v_main = (V // _NS) // 128 * 128
    v_tail = V - v_main * _NS

    mesh = plsc.VectorSubcoreMesh(
        core_axis_name="c", subcore_axis_name="s",
        num_cores=_NC, num_subcores=_NS,
    )

    @functools.partial(
        pl.kernel,
        out_type=jax.ShapeDtypeStruct((H, D, B), jnp.float32),
        mesh=mesh,
        scratch_types=[
            pltpu.VMEM((cs,), jnp.int32),
            pltpu.VMEM((cs,), jnp.int32),
            pltpu.VMEM((cs,), jnp.float32),
            pltpu.VMEM((cs,), jnp.float32),
            pltpu.VMEM_SHARED((1, V), jnp.float32),
            pltpu.VMEM_SHARED((1, V), jnp.float32),
            pltpu.SemaphoreType.DMA,
            pltpu.SemaphoreType.DMA,
            pltpu.SemaphoreType.DMA,
            pltpu.SemaphoreType.DMA,
            pltpu.SemaphoreType.DMA,
            pltpu.SemaphoreType.DMA,
        ],
    )
    def k(idx_hbm, table_hbm, out_hbm, idx0, idx1, val0, val1, row_a, row_b,
          ssem_a, ssem_b, isem, gsem, wsem0, wsem1):
        c = lax.axis_index("c")
        s = lax.axis_index("s")
        b0 = pl.multiple_of(s * b_chunk, 128)
        v0 = pl.multiple_of(s * v_main, 128)
        d_base = c * d_per_core
        idxb = (idx0, idx1)
        valb = (val0, val1)
        wsem = (wsem0, wsem1)

        def idx_copy(q, buf):
            return pltpu.make_async_copy(
                idx_hbm.at[s, pl.ds((q % nq) * cs, cs)], buf, isem)

        def stage_descrs(row, row_sh, ssem):
            return (
                pltpu.make_async_copy(
                    table_hbm.at[pl.ds(row, 1), pl.ds(v0, v_main)],
                    row_sh.at[:, pl.ds(v0, v_main)], ssem),
                pltpu.make_async_copy(
                    table_hbm.at[pl.ds(row, 1), pl.ds(v_main * _NS, v_tail)],
                    row_sh.at[:, pl.ds(v_main * _NS, v_tail)], ssem),
            )

        def fire_stage(row, row_sh, ssem):
            ds_ = stage_descrs(row, row_sh, ssem)
            ds_[0].start()
            @pl.when(s == _NS - 1)
            def _():
                ds_[1].start()

        def wait_stage(row, row_sh, ssem):
            ds_ = stage_descrs(row, row_sh, ssem)
            ds_[0].wait()
            @pl.when(s == _NS - 1)
            def _():
                ds_[1].wait()

        def drain(sem, buf):
            pltpu.make_async_copy(table_hbm.at[0, pl.ds(0, cs)], buf,
                                  sem).wait()

        def fire_writes(q, row):
            p = q % 2
            for i in range(_KH):
                h = q * _KH + i
                pltpu.make_async_copy(
                    valb[p].at[pl.ds(i * b_chunk, b_chunk)],
                    out_hbm.at[h, row, pl.ds(b0, b_chunk)], wsem[p]).start()

        def process(d, row_sh, other_sh, ssem, other_ssem):
            row = d_base + d
            plsc.subcore_barrier()
            @pl.when(d + 1 < d_per_core)
            def _():
                fire_stage(row + 1, other_sh, other_ssem)
            wait_stage(row, row_sh, ssem)
            plsc.subcore_barrier()
            for q in range(nq):
                p = q % 2
                if q >= 2:
                    drain(wsem[p], valb[p])
                else:
                    @pl.when(d > 0)
                    def _():
                        drain(wsem[p], valb[p])
                idx_copy(q, idxb[p]).wait()
                pltpu.make_async_copy(row_sh.at[0].at[idxb[p]], valb[p],
                                      gsem).start()
                if q > 0:
                    drain(gsem, valb[1 - p])
                idx_copy(q + 1, idxb[(q + 1) % 2]).start()
                if q > 0:
                    fire_writes(q - 1, row)
            drain(gsem, valb[(nq - 1) % 2])
            fire_writes(nq - 1, row)

        idx_copy(0, idxb[0]).start()
        fire_stage(d_base, row_a, ssem_a)

        def body(i, carry):
            d = i * 2
            process(d, row_a, row_b, ssem_a, ssem_b)
            process(d + 1, row_b, row_a, ssem_b, ssem_a)
            return carry

        lax.fori_loop(0, d_per_core // 2, body, 0)
        idx_copy(0, idxb[0]).wait()
        drain(wsem[0], valb[0])
        drain(wsem[1], valb[1])

    return k(idx_tiles, table_t)


def kernel(indices, lookup_table):
    B0, H = indices.shape
    idx_tiles = (indices.T.reshape(H, _NS, B0 // _NS)
                 .transpose(1, 0, 2).reshape(_NS, H * (B0 // _NS)))
    out3 = _gather(idx_tiles, lookup_table.T, H, B0)
    return out3.transpose(2, 0, 1)


# speedup vs baseline: 1.0092x; 1.0092x over previous
import functools

import jax
import jax.numpy as jnp
from jax import lax
from jax.experimental import pallas as pl
from jax.experimental.pallas import tpu as pltpu
from jax.experimental.pallas import tpu_sc as plsc

_NC = 2
_NS = 16
_KH = 5


@functools.partial(jax.jit, static_argnames=("H", "B"))
def _gather(idx_tiles, table_t, H, B):
    D, V = table_t.shape
    d_per_core = D // _NC
    b_chunk = B // _NS
    nq = H // _KH
    cs = _KH * b_chunk
    v_main = (V // _NS) // 128 * 128
    v_tail = V - v_main * _NS

    mesh = plsc.VectorSubcoreMesh(
        core_axis_name="c", subcore_axis_name="s",
        num_cores=_NC, num_subcores=_NS,
    )

    @functools.partial(
        pl.kernel,
        out_type=jax.ShapeDtypeStruct((H, D, B), jnp.float32),
        mesh=mesh,
        scratch_types=[
            pltpu.VMEM((cs,), jnp.int32),
            pltpu.VMEM((cs,), jnp.int32),
            pltpu.VMEM((cs,), jnp.float32),
            pltpu.VMEM((cs,), jnp.float32),
            pltpu.VMEM_SHARED((1, V), jnp.float32),
            pltpu.VMEM_SHARED((1, V), jnp.float32),
            pltpu.SemaphoreType.DMA,
            pltpu.SemaphoreType.DMA,
            pltpu.SemaphoreType.DMA,
            pltpu.SemaphoreType.DMA,
            pltpu.SemaphoreType.DMA,
            pltpu.SemaphoreType.DMA,
        ],
    )
    def k(idx_hbm, table_hbm, out_hbm, idx0, idx1, val0, val1, row_a, row_b,
          ssem_a, ssem_b, isem, gsem, wsem0, wsem1):
        c = lax.axis_index("c")
        s = lax.axis_index("s")
        b0 = pl.multiple_of(s * b_chunk, 128)
        v0 = pl.multiple_of(s * v_main, 128)
        d_base = c * d_per_core
        idxb = (idx0, idx1)
        valb = (val0, val1)
        wsem = (wsem0, wsem1)

        def idx_copy(q, buf):
            return pltpu.make_async_copy(
                idx_hbm.at[s, pl.ds((q % nq) * cs, cs)], buf, isem)

        def stage_descrs(row, row_sh, ssem):
            return (
                pltpu.make_async_copy(
                    table_hbm.at[pl.ds(row, 1), pl.ds(v0, v_main)],
                    row_sh.at[:, pl.ds(v0, v_main)], ssem),
                pltpu.make_async_copy(
                    table_hbm.at[pl.ds(row, 1), pl.ds(v_main * _NS, v_tail)],
                    row_sh.at[:, pl.ds(v_main * _NS, v_tail)], ssem),
            )

        def fire_stage(row, row_sh, ssem):
            ds_ = stage_descrs(row, row_sh, ssem)
            ds_[0].start()
            @pl.when(s == _NS - 1)
            def _():
                ds_[1].start()

        def wait_stage(row, row_sh, ssem):
            ds_ = stage_descrs(row, row_sh, ssem)
            ds_[0].wait()
            @pl.when(s == _NS - 1)
            def _():
                ds_[1].wait()

        def drain(sem, buf):
            pltpu.make_async_copy(table_hbm.at[0, pl.ds(0, cs)], buf,
                                  sem).wait()

        def fire_writes(q, row):
            p = q % 2
            for i in range(_KH):
                h = q * _KH + i
                pltpu.make_async_copy(
                    valb[p].at[pl.ds(i * b_chunk, b_chunk)],
                    out_hbm.at[h, row, pl.ds(b0, b_chunk)], wsem[p]).start()

        def process(d, row_sh, other_sh, ssem, other_ssem):
            row = d_base + d
            wait_stage(row, row_sh, ssem)
            plsc.subcore_barrier()
            @pl.when(d + 1 < d_per_core)
            def _():
                fire_stage(row + 1, other_sh, other_ssem)
            for q in range(nq):
                p = q % 2
                if q >= 2:
                    drain(wsem[p], valb[p])
                else:
                    @pl.when(d > 0)
                    def _():
                        drain(wsem[p], valb[p])
                idx_copy(q, idxb[p]).wait()
                pltpu.make_async_copy(row_sh.at[0].at[idxb[p]], valb[p],
                                      gsem).start()
                if q > 0:
                    drain(gsem, valb[1 - p])
                idx_copy(q + 1, idxb[(q + 1) % 2]).start()
                if q > 0:
                    fire_writes(q - 1, row)
            drain(gsem, valb[(nq - 1) % 2])
            fire_writes(nq - 1, row)

        idx_copy(0, idxb[0]).start()
        fire_stage(d_base, row_a, ssem_a)

        def body(i, carry):
            d = i * 2
            process(d, row_a, row_b, ssem_a, ssem_b)
            process(d + 1, row_b, row_a, ssem_b, ssem_a)
            return carry

        lax.fori_loop(0, d_per_core // 2, body, 0)
        idx_copy(0, idxb[0]).wait()
        drain(wsem[0], valb[0])
        drain(wsem[1], valb[1])

    return k(idx_tiles, table_t)


def kernel(indices, lookup_table):
    B0, H = indices.shape
    idx_tiles = (indices.T.reshape(H, _NS, B0 // _NS)
                 .transpose(1, 0, 2).reshape(_NS, H * (B0 // _NS)))
    out3 = _gather(idx_tiles, lookup_table.T, H, B0)
    return out3.transpose(2, 0, 1)
